# Initial kernel scaffold; baseline (speedup 1.0000x reference)
import jax
import jax.numpy as jnp
from jax.experimental import pallas as pl


def kernel(labels, noise, class_means, class_stds):
    raise NotImplementedError("write your pallas kernel here")


# speedup vs baseline: 2.3178x; 2.3178x over previous
import functools
import jax
import jax.numpy as jnp
from jax import lax
from jax.experimental import pallas as pl
from jax.experimental.pallas import tpu as pltpu
from jax.experimental.pallas import tpu_sc as plsc

NUM_CLASSES = 1000
C, H, W = 3, 32, 32
D = C * H * W
BATCH = 16384
NC, NS = 2, 16
NW = NC * NS
BPW = BATCH // NW
K = 8
NCHUNK = BPW // K
LANES = 16
COLS = D // LANES


def _sc_body(means_hbm, stds_hbm, labels_hbm, noise_hbm, out_hbm,
             idx_v, mean_v, std_v, noise_v,
             sem_m, sem_s, sem_n, sem_o):
    wid = lax.axis_index("s") * NC + lax.axis_index("c")
    base = wid * BPW

    pltpu.sync_copy(labels_hbm.at[wid], idx_v)

    def chunk(c, carry):
        row0 = base + c * K
        cm = pltpu.async_copy(means_hbm.at[idx_v.at[c]], mean_v, sem_m)
        cs = pltpu.async_copy(stds_hbm.at[idx_v.at[c]], std_v, sem_s)
        cn = pltpu.async_copy(noise_hbm.at[pl.ds(row0, K)], noise_v, sem_n)
        cm.wait()
        cs.wait()
        cn.wait()

        def col(j, carry2):
            off = j * LANES
            for k in range(K):
                n = noise_v[k, pl.ds(off, LANES)]
                m = mean_v[k, pl.ds(off, LANES)]
                s = std_v[k, pl.ds(off, LANES)]
                noise_v[k, pl.ds(off, LANES)] = m + s * n
            return carry2

        lax.fori_loop(0, COLS, col, 0)
        pltpu.async_copy(noise_v, out_hbm.at[pl.ds(row0, K)], sem_o).wait()
        return carry

    lax.fori_loop(0, NCHUNK, chunk, 0)


@functools.partial(
    pl.kernel,
    out_type=jax.ShapeDtypeStruct((BATCH, D), jnp.float32),
    mesh=plsc.VectorSubcoreMesh(
        core_axis_name="c", subcore_axis_name="s",
        num_cores=NC, num_subcores=NS),
    scratch_types=[
        pltpu.VMEM((NCHUNK, K), jnp.int32),
        pltpu.VMEM((K, D), jnp.float32),
        pltpu.VMEM((K, D), jnp.float32),
        pltpu.VMEM((K, D), jnp.float32),
        pltpu.SemaphoreType.DMA,
        pltpu.SemaphoreType.DMA,
        pltpu.SemaphoreType.DMA,
        pltpu.SemaphoreType.DMA,
    ],
)
def _sc_embed(means_hbm, stds_hbm, labels_hbm, noise_hbm, out_hbm,
              idx_v, mean_v, std_v, noise_v,
              sem_m, sem_s, sem_n, sem_o):
    _sc_body(means_hbm, stds_hbm, labels_hbm, noise_hbm, out_hbm,
             idx_v, mean_v, std_v, noise_v,
             sem_m, sem_s, sem_n, sem_o)


@jax.jit
def kernel(labels, noise, class_means, class_stds):
    means2 = class_means.reshape(NUM_CLASSES, D)
    stds2 = class_stds.reshape(NUM_CLASSES, D)
    noise2 = noise.reshape(BATCH, D)
    labels3 = labels.reshape(NW, NCHUNK, K)
    out = _sc_embed(means2, stds2, labels3, noise2)
    return out.reshape(BATCH, C, H, W)


# speedup vs baseline: 3.3806x; 1.4585x over previous
import functools
import jax
import jax.numpy as jnp
from jax import lax
from jax.experimental import pallas as pl
from jax.experimental.pallas import tpu as pltpu
from jax.experimental.pallas import tpu_sc as plsc

NUM_CLASSES = 1000
C, H, W = 3, 32, 32
D = C * H * W
BATCH = 16384
NC, NS = 2, 16
NW = NC * NS
BPW = BATCH // NW
K = 4
NCHUNK = BPW // K
NBUF = 2
LANES = 16
COLS = D // LANES


def _sc_body(means_hbm, stds_hbm, labels_hbm, noise_hbm, out_hbm,
             idx_v, mean_v, std_v, noise_v, out_v, sem_in, sem_out):
    wid = lax.axis_index("s") * NC + lax.axis_index("c")
    base = wid * BPW

    pltpu.sync_copy(labels_hbm.at[wid], idx_v)

    def start_in(b, c):
        row0 = base + c * K
        pltpu.async_copy(means_hbm.at[idx_v.at[c]], mean_v[b], sem_in[b])
        pltpu.async_copy(stds_hbm.at[idx_v.at[c]], std_v[b], sem_in[b])
        pltpu.async_copy(noise_hbm.at[pl.ds(row0, K)], noise_v[b], sem_in[b])

    def wait_in(b):
        pltpu.make_async_copy(means_hbm.at[idx_v.at[0]], mean_v[b],
                              sem_in[b]).wait()
        pltpu.make_async_copy(stds_hbm.at[idx_v.at[0]], std_v[b],
                              sem_in[b]).wait()
        pltpu.make_async_copy(noise_hbm.at[pl.ds(base, K)], noise_v[b],
                              sem_in[b]).wait()

    def start_out(b, c):
        row0 = base + c * K
        pltpu.async_copy(out_v[b], out_hbm.at[pl.ds(row0, K)], sem_out[b])

    def wait_out(b):
        pltpu.make_async_copy(out_v[b], out_hbm.at[pl.ds(base, K)],
                              sem_out[b]).wait()

    for b in range(NBUF):
        start_in(b, b)

    def iteration(i, carry):
        for b in range(NBUF):
            cc = i * NBUF + b
            wait_in(b)

            @pl.when(cc >= NBUF)
            def _():
                wait_out(b)

            def col(j, carry2):
                off = j * LANES
                for k in range(K):
                    n = noise_v[b][k, pl.ds(off, LANES)]
                    m = mean_v[b][k, pl.ds(off, LANES)]
                    s = std_v[b][k, pl.ds(off, LANES)]
                    out_v[b][k, pl.ds(off, LANES)] = m + s * n
                return carry2

            lax.fori_loop(0, COLS, col, 0)
            start_out(b, cc)

            @pl.when(cc + NBUF < NCHUNK)
            def _():
                start_in(b, cc + NBUF)
        return carry

    lax.fori_loop(0, NCHUNK // NBUF, iteration, 0)
    for b in range(NBUF):
        wait_out(b)


@functools.partial(
    pl.kernel,
    out_type=jax.ShapeDtypeStruct((BATCH, D), jnp.float32),
    mesh=plsc.VectorSubcoreMesh(
        core_axis_name="c", subcore_axis_name="s",
        num_cores=NC, num_subcores=NS),
    scratch_types=[
        pltpu.VMEM((NCHUNK, K), jnp.int32),
        [pltpu.VMEM((K, D), jnp.float32) for _ in range(NBUF)],
        [pltpu.VMEM((K, D), jnp.float32) for _ in range(NBUF)],
        [pltpu.VMEM((K, D), jnp.float32) for _ in range(NBUF)],
        [pltpu.VMEM((K, D), jnp.float32) for _ in range(NBUF)],
        [pltpu.SemaphoreType.DMA for _ in range(NBUF)],
        [pltpu.SemaphoreType.DMA for _ in range(NBUF)],
    ],
)
def _sc_embed(means_hbm, stds_hbm, labels_hbm, noise_hbm, out_hbm,
              idx_v, mean_v, std_v, noise_v, out_v, sem_in, sem_out):
    _sc_body(means_hbm, stds_hbm, labels_hbm, noise_hbm, out_hbm,
             idx_v, mean_v, std_v, noise_v, out_v, sem_in, sem_out)


@jax.jit
def kernel(labels, noise, class_means, class_stds):
    means2 = class_means.reshape(NUM_CLASSES, D)
    stds2 = class_stds.reshape(NUM_CLASSES, D)
    noise2 = noise.reshape(BATCH, D)
    labels3 = labels.reshape(NW, NCHUNK, K)
    out = _sc_embed(means2, stds2, labels3, noise2)
    return out.reshape(BATCH, C, H, W)
